# baseline (speedup 1.0000x reference)
import functools

import jax
import jax.numpy as jnp
from jax.experimental import pallas as pl
from jax.experimental.pallas import tpu as pltpu

KS = 5
KT = KS * KS
KPAD = 32
C = 128
TE = 256
NC = 256
CORES = 2
DEG_LANE = 127
VMEM_LIMIT = 48 * 1024 * 1024


def _round_up(x, m):
    return ((x + m - 1) // m) * m


def _layer_kernel(dst_ref, hs_ref, basis_ref, w_ref, o_ref, xsk_ref,
                  *, cin, kc, kc_pad, nchunks):
    e = pl.program_id(1)

    @pl.when(e == 0)
    def _():
        o_ref[...] = jnp.zeros_like(o_ref)
        if kc_pad > kc:
            xsk_ref[:, kc:] = jnp.zeros((TE, kc_pad - kc), xsk_ref.dtype)

    basis = basis_ref[...]
    hs = hs_ref[:, :cin]
    for k in range(KT):
        xsk_ref[:, k * cin:(k + 1) * cin] = basis[:, k:k + 1] * hs

    m = jnp.dot(xsk_ref[...], w_ref[...], preferred_element_type=jnp.float32)
    lane = jax.lax.broadcasted_iota(jnp.int32, (TE, C), 1)
    msgs = (m + (lane == DEG_LANE).astype(jnp.float32)).astype(jnp.bfloat16)

    dst = dst_ref[0]
    rows = jax.lax.broadcasted_iota(jnp.int32, (NC, TE), 0)
    for c in range(nchunks):
        oh = (rows + c * NC == dst).astype(jnp.bfloat16)
        o_ref[0, c * NC:(c + 1) * NC, :] += jnp.dot(
            oh, msgs, preferred_element_type=jnp.float32)


def _finish(acc0, acc1, h, root, bias, cout):
    s = acc0 + acc1
    inv = 1.0 / jnp.maximum(s[:, DEG_LANE:DEG_LANE + 1], 1.0)
    a = (s * inv
         + jnp.dot(h, root, preferred_element_type=jnp.float32)
         + bias)
    elu = jnp.where(a > 0.0, a, jnp.exp(jnp.minimum(a, 0.0)) - 1.0)
    lane = jax.lax.broadcasted_iota(jnp.int32, elu.shape, 1)
    return jnp.where(lane < cout, elu, 0.0)


def _combine_kernel(acc_ref, h_ref, root_ref, bias_ref, o_ref, *, cout):
    o_ref[...] = _finish(acc_ref[0], acc_ref[1], h_ref[...], root_ref[...],
                         bias_ref[...], cout).astype(jnp.bfloat16)


def _finale_kernel(acc_ref, h_ref, root_ref, bias_ref, pool_ref, wfc_ref,
                   bfc_ref, o_ref, *, cout):
    h3 = _finish(acc_ref[0], acc_ref[1], h_ref[...], root_ref[...],
                 bias_ref[...], cout).astype(jnp.bfloat16)
    pooled = jnp.dot(pool_ref[...], h3, preferred_element_type=jnp.float32)
    logits = jnp.dot(pooled.astype(jnp.bfloat16), wfc_ref[...],
                     preferred_element_type=jnp.float32) + bfc_ref[...]
    mx = jnp.max(logits, axis=1, keepdims=True)
    z = logits - mx
    o_ref[...] = z - jnp.log(jnp.sum(jnp.exp(z), axis=1, keepdims=True))


_VMEM_FULL = pl.BlockSpec(memory_space=pltpu.MemorySpace.VMEM)


def _accumulate(hs, basis, dst3, weight, cin, n_pad, e_pad):
    cout = weight.shape[2]
    kc = KT * cin
    kc_pad = _round_up(kc, 128)
    w_flat = jnp.pad(weight.reshape(kc, cout),
                     ((0, kc_pad - kc), (0, C - cout))).astype(jnp.bfloat16)
    etc = e_pad // TE // CORES
    nchunks = n_pad // NC

    return pl.pallas_call(
        functools.partial(_layer_kernel, cin=cin, kc=kc, kc_pad=kc_pad,
                          nchunks=nchunks),
        out_shape=jax.ShapeDtypeStruct((CORES, n_pad, C), jnp.float32),
        grid=(CORES, etc),
        in_specs=[
            pl.BlockSpec((1, 1, TE), lambda c, e: (c * etc + e, 0, 0)),
            pl.BlockSpec((TE, C), lambda c, e: (c * etc + e, 0)),
            pl.BlockSpec((TE, KPAD), lambda c, e: (c * etc + e, 0)),
            pl.BlockSpec((kc_pad, C), lambda c, e: (0, 0)),
        ],
        out_specs=pl.BlockSpec((1, n_pad, C), lambda c, e: (c, 0, 0)),
        scratch_shapes=[pltpu.VMEM((TE, kc_pad), jnp.bfloat16)],
        compiler_params=pltpu.CompilerParams(
            dimension_semantics=("parallel", "arbitrary"),
            vmem_limit_bytes=VMEM_LIMIT),
    )(dst3, hs, basis, w_flat)


def _combine(acc, h, root, bias, cin, cout, n_pad):
    root_p = jnp.pad(root, ((0, C - cin), (0, C - cout))).astype(jnp.bfloat16)
    bias_p = jnp.pad(bias, ((0, 0), (0, C - cout)))
    hn = n_pad // CORES
    return pl.pallas_call(
        functools.partial(_combine_kernel, cout=cout),
        out_shape=jax.ShapeDtypeStruct((n_pad, C), jnp.bfloat16),
        grid=(CORES,),
        in_specs=[
            pl.BlockSpec((CORES, hn, C), lambda i: (0, i, 0)),
            pl.BlockSpec((hn, C), lambda i: (i, 0)),
            pl.BlockSpec((C, C), lambda i: (0, 0)),
            pl.BlockSpec((1, C), lambda i: (0, 0)),
        ],
        out_specs=pl.BlockSpec((hn, C), lambda i: (i, 0)),
        compiler_params=pltpu.CompilerParams(
            dimension_semantics=("parallel",),
            vmem_limit_bytes=VMEM_LIMIT),
    )(acc, h, root_p, bias_p)


def _finale(acc, h, root, bias, pool, w_fc, b_fc, cin, cout):
    root_p = jnp.pad(root, ((0, C - cin), (0, C - cout))).astype(jnp.bfloat16)
    bias_p = jnp.pad(bias, ((0, 0), (0, C - cout)))
    w_fc_p = jnp.pad(w_fc, ((0, C - w_fc.shape[0]), (0, 0))).astype(jnp.bfloat16)
    G = pool.shape[0]
    n_cls = w_fc.shape[1]
    return pl.pallas_call(
        functools.partial(_finale_kernel, cout=cout),
        out_shape=jax.ShapeDtypeStruct((G, n_cls), jnp.float32),
        in_specs=[_VMEM_FULL] * 7,
        out_specs=_VMEM_FULL,
        compiler_params=pltpu.CompilerParams(vmem_limit_bytes=VMEM_LIMIT),
    )(acc, h, root_p, bias_p, pool, w_fc_p, b_fc)


def _spline_basis(pseudo):
    v = jnp.clip(pseudo.astype(jnp.float32), 0.0, 1.0) * (KS - 1)
    k0 = jnp.clip(jnp.floor(v), 0.0, KS - 2)
    frac = v - k0
    k0 = k0.astype(jnp.int32)
    B = jnp.zeros((pseudo.shape[0], KT), jnp.float32)
    for s0 in (0, 1):
        for s1 in (0, 1):
            c0 = frac[:, 0] if s0 else (1.0 - frac[:, 0])
            c1 = frac[:, 1] if s1 else (1.0 - frac[:, 1])
            idx = (k0[:, 0] + s0) + KS * (k0[:, 1] + s1)
            B = B + (c0 * c1)[:, None] * jax.nn.one_hot(
                idx, KT, dtype=jnp.float32)
    return B


@functools.partial(jax.jit, static_argnames=("num_graphs",))
def _forward(params, x, edge_index, pseudo, batch, num_graphs):
    N = x.shape[0]
    E = edge_index.shape[1]
    src, dst = edge_index[0], edge_index[1]

    n_pad = _round_up(N, NC * CORES)
    e_pad = _round_up(E, TE * CORES)

    basis = _spline_basis(pseudo)
    basis = jnp.pad(basis, ((0, e_pad - E), (0, KPAD - KT))
                    ).astype(jnp.bfloat16)
    dst3 = jnp.concatenate(
        [dst, jnp.full((e_pad - E,), -1, jnp.int32)]
    ).reshape(e_pad // TE, 1, TE)
    src_p = jnp.concatenate([src, jnp.zeros((e_pad - E,), jnp.int32)])

    g_ids = jax.lax.broadcasted_iota(jnp.int32, (num_graphs, N), 0)
    pool = (g_ids == batch[None, :]).astype(jnp.float32)
    pool = pool / jnp.maximum(jnp.sum(pool, axis=1, keepdims=True), 1.0)
    pool = jnp.pad(pool, ((0, 0), (0, n_pad - N))).astype(jnp.bfloat16)

    h = jnp.zeros((n_pad, C), jnp.bfloat16).at[:N, :x.shape[1]].set(
        x.astype(jnp.bfloat16))

    dims = (("conv1", 8, 32), ("conv2", 32, 64), ("conv3", 64, 64))
    out = None
    for name, cin, cout in dims:
        p = params[name]
        hs = jnp.tile(h[:256], (e_pad // 256, 1))
        acc = _accumulate(hs, basis, dst3, p["weight"], cin, n_pad, e_pad)
        if name != "conv3":
            h = _combine(acc, h, p["root"], p["bias"], cin, cout, n_pad)
        else:
            out = _finale(acc, h, p["root"], p["bias"], pool,
                          params["fc1"]["weight"], params["fc1"]["bias"],
                          cin, cout)
    return out


def kernel(x, edge_index, pseudo, batch,
           conv1_weight, conv1_root, conv1_bias,
           conv2_weight, conv2_root, conv2_bias,
           conv3_weight, conv3_root, conv3_bias,
           fc1_weight, fc1_bias):
    params = {
        "conv1": {"weight": conv1_weight, "root": conv1_root, "bias": conv1_bias},
        "conv2": {"weight": conv2_weight, "root": conv2_root, "bias": conv2_bias},
        "conv3": {"weight": conv3_weight, "root": conv3_root, "bias": conv3_bias},
        "fc1":   {"weight": fc1_weight, "bias": fc1_bias},
    }
    return _forward(params, x, edge_index, pseudo, batch, num_graphs=64)


# speedup vs baseline: 2.8236x; 2.8236x over previous
import functools

import jax
import jax.numpy as jnp
from jax.experimental import pallas as pl
from jax.experimental.pallas import tpu as pltpu

KS = 5
KT = KS * KS
KPAD = 32
C = 128
TE = 256
CORES = 2
DEG_ROW = 127
ONE_ROW = 126
VMEM_LIMIT = 48 * 1024 * 1024


def _round_up(x, m):
    return ((x + m - 1) // m) * m


def _layer_kernel(src3_ref, idx_ref, basisT_ref, hT_ref, wT_ref, o_ref,
                  xskT_ref, *, cin, kc, kc_pad, n_pad):
    e = pl.program_id(1)

    @pl.when(e == 0)
    def _():
        o_ref[...] = jnp.zeros_like(o_ref)
        if kc_pad > kc:
            xskT_ref[kc:, :] = jnp.zeros((kc_pad - kc, TE), xskT_ref.dtype)

    src = src3_ref[0]
    rows_n = jax.lax.broadcasted_iota(jnp.int32, (n_pad, TE), 0)
    ohs = (rows_n == src).astype(jnp.bfloat16)
    hsT = jnp.dot(hT_ref[...], ohs,
                  preferred_element_type=jnp.float32).astype(jnp.bfloat16)

    basisT = basisT_ref[...]
    hsc = hsT[:cin, :]
    for k in range(KT):
        xskT_ref[k * cin:(k + 1) * cin, :] = basisT[k:k + 1, :] * hsc

    m = jnp.dot(wT_ref[...], xskT_ref[...],
                preferred_element_type=jnp.float32)
    rows_c = jax.lax.broadcasted_iota(jnp.int32, (C, TE), 0)
    msgsT = (m + (rows_c == DEG_ROW).astype(jnp.float32)).astype(jnp.bfloat16)

    dstc = idx_ref[0][:, 0:1]
    lanes = jax.lax.broadcasted_iota(jnp.int32, (TE, n_pad), 1)
    ohd = (lanes == dstc).astype(jnp.bfloat16)
    o_ref[0] += jnp.dot(msgsT, ohd, preferred_element_type=jnp.float32)


def _finishT(acc0, acc1, hT, rootT, cout):
    s = acc0 + acc1
    inv = 1.0 / jnp.maximum(s[DEG_ROW:DEG_ROW + 1, :], 1.0)
    a = s * inv + jnp.dot(rootT, hT, preferred_element_type=jnp.float32)
    elu = jnp.where(a > 0.0, a, jnp.exp(jnp.minimum(a, 0.0)) - 1.0)
    rows = jax.lax.broadcasted_iota(jnp.int32, elu.shape, 0)
    out = jnp.where(rows < cout, elu, 0.0)
    return jnp.where(rows == ONE_ROW, 1.0, out)


def _combine_kernel(acc_ref, hT_ref, rootT_ref, o_ref, *, cout):
    o_ref[...] = _finishT(acc_ref[0], acc_ref[1], hT_ref[...],
                          rootT_ref[...], cout).astype(jnp.bfloat16)


def _finale_kernel(acc_ref, hT_ref, rootT_ref, poolT_ref, wfc_ref, bfc_ref,
                   o_ref, *, cout):
    h3 = _finishT(acc_ref[0], acc_ref[1], hT_ref[...], rootT_ref[...],
                  cout).astype(jnp.bfloat16)
    pooledT = jnp.dot(h3, poolT_ref[...],
                      preferred_element_type=jnp.float32)
    logits = jax.lax.dot_general(
        pooledT.astype(jnp.bfloat16), wfc_ref[...],
        (((0,), (0,)), ((), ())),
        preferred_element_type=jnp.float32) + bfc_ref[...]
    mx = jnp.max(logits, axis=1, keepdims=True)
    z = logits - mx
    o_ref[...] = z - jnp.log(jnp.sum(jnp.exp(z), axis=1, keepdims=True))


_VMEM_FULL = pl.BlockSpec(memory_space=pltpu.MemorySpace.VMEM)


def _accumulate(hT, basisT, src3, idx_pack, weight, cin, n_pad, e_pad):
    cout = weight.shape[2]
    kc = KT * cin
    kc_pad = _round_up(kc, 8)
    wT = jnp.pad(weight.reshape(kc, cout).T,
                 ((0, C - cout), (0, kc_pad - kc))).astype(jnp.bfloat16)
    etc = e_pad // TE // CORES

    return pl.pallas_call(
        functools.partial(_layer_kernel, cin=cin, kc=kc, kc_pad=kc_pad,
                          n_pad=n_pad),
        out_shape=jax.ShapeDtypeStruct((CORES, C, n_pad), jnp.float32),
        grid=(CORES, etc),
        in_specs=[
            pl.BlockSpec((1, 1, TE), lambda c, e: (c * etc + e, 0, 0)),
            pl.BlockSpec((1, TE, 8), lambda c, e: (c * etc + e, 0, 0)),
            pl.BlockSpec((KPAD, TE), lambda c, e: (0, c * etc + e)),
            pl.BlockSpec((C, n_pad), lambda c, e: (0, 0)),
            pl.BlockSpec((C, kc_pad), lambda c, e: (0, 0)),
        ],
        out_specs=pl.BlockSpec((1, C, n_pad), lambda c, e: (c, 0, 0)),
        scratch_shapes=[pltpu.VMEM((kc_pad, TE), jnp.bfloat16)],
        compiler_params=pltpu.CompilerParams(
            dimension_semantics=("parallel", "arbitrary"),
            vmem_limit_bytes=VMEM_LIMIT),
    )(src3, idx_pack, basisT, hT, wT)


def _root_aug(root, bias, cin, cout):
    rootT = jnp.zeros((C, C), jnp.float32)
    rootT = rootT.at[:cout, :cin].set(root.T)
    rootT = rootT.at[:cout, ONE_ROW].set(bias[0])
    return rootT.astype(jnp.bfloat16)


def _combine(acc, hT, root, bias, cin, cout, n_pad):
    hn = n_pad // CORES
    return pl.pallas_call(
        functools.partial(_combine_kernel, cout=cout),
        out_shape=jax.ShapeDtypeStruct((C, n_pad), jnp.bfloat16),
        grid=(CORES,),
        in_specs=[
            pl.BlockSpec((CORES, C, hn), lambda i: (0, 0, i)),
            pl.BlockSpec((C, hn), lambda i: (0, i)),
            pl.BlockSpec((C, C), lambda i: (0, 0)),
        ],
        out_specs=pl.BlockSpec((C, hn), lambda i: (0, i)),
        compiler_params=pltpu.CompilerParams(
            dimension_semantics=("parallel",),
            vmem_limit_bytes=VMEM_LIMIT),
    )(acc, hT, _root_aug(root, bias, cin, cout))


def _finale(acc, hT, root, bias, poolT, w_fc, b_fc, cin, cout):
    w_fc_p = jnp.pad(w_fc, ((0, C - w_fc.shape[0]), (0, 0))).astype(jnp.bfloat16)
    G = poolT.shape[1]
    n_cls = w_fc.shape[1]
    return pl.pallas_call(
        functools.partial(_finale_kernel, cout=cout),
        out_shape=jax.ShapeDtypeStruct((G, n_cls), jnp.float32),
        in_specs=[_VMEM_FULL] * 6,
        out_specs=_VMEM_FULL,
        compiler_params=pltpu.CompilerParams(vmem_limit_bytes=VMEM_LIMIT),
    )(acc, hT, _root_aug(root, bias, cin, cout), poolT, w_fc_p, b_fc)


def _spline_basis(pseudo):
    v = jnp.clip(pseudo.astype(jnp.float32), 0.0, 1.0) * (KS - 1)
    k0 = jnp.clip(jnp.floor(v), 0.0, KS - 2)
    frac = v - k0
    k0 = k0.astype(jnp.int32)
    B = jnp.zeros((pseudo.shape[0], KT), jnp.float32)
    for s0 in (0, 1):
        for s1 in (0, 1):
            c0 = frac[:, 0] if s0 else (1.0 - frac[:, 0])
            c1 = frac[:, 1] if s1 else (1.0 - frac[:, 1])
            idx = (k0[:, 0] + s0) + KS * (k0[:, 1] + s1)
            B = B + (c0 * c1)[:, None] * jax.nn.one_hot(
                idx, KT, dtype=jnp.float32)
    return B


@functools.partial(jax.jit, static_argnames=("num_graphs",))
def _forward(params, x, edge_index, pseudo, batch, num_graphs):
    N = x.shape[0]
    E = edge_index.shape[1]
    src, dst = edge_index[0], edge_index[1]

    n_pad = _round_up(N, 128 * CORES)
    e_pad = _round_up(E, TE * CORES)

    basisT = jnp.pad(_spline_basis(pseudo),
                     ((0, e_pad - E), (0, KPAD - KT))).astype(jnp.bfloat16).T

    dst_p = jnp.concatenate([dst, jnp.full((e_pad - E,), -1, jnp.int32)])
    src_p = jnp.concatenate([src, jnp.zeros((e_pad - E,), jnp.int32)])
    src3 = src_p.reshape(e_pad // TE, 1, TE)
    idx_pack = jnp.zeros((e_pad, 8), jnp.int32).at[:, 0].set(dst_p)
    idx_pack = idx_pack.reshape(e_pad // TE, TE, 8)

    g_ids = jax.lax.broadcasted_iota(jnp.int32, (num_graphs, N), 0)
    pool = (g_ids == batch[None, :]).astype(jnp.float32)
    pool = pool / jnp.maximum(jnp.sum(pool, axis=1, keepdims=True), 1.0)
    poolT = jnp.pad(pool, ((0, 0), (0, n_pad - N))).astype(jnp.bfloat16).T

    hT = jnp.zeros((C, n_pad), jnp.bfloat16)
    hT = hT.at[:x.shape[1], :N].set(x.T.astype(jnp.bfloat16))
    hT = hT.at[ONE_ROW, :].set(jnp.bfloat16(1.0))

    dims = (("conv1", 8, 32), ("conv2", 32, 64), ("conv3", 64, 64))
    out = None
    for name, cin, cout in dims:
        p = params[name]
        acc = _accumulate(hT, basisT, src3, idx_pack, p["weight"],
                          cin, n_pad, e_pad)
        if name != "conv3":
            hT = _combine(acc, hT, p["root"], p["bias"], cin, cout, n_pad)
        else:
            out = _finale(acc, hT, p["root"], p["bias"], poolT,
                          params["fc1"]["weight"], params["fc1"]["bias"],
                          cin, cout)
    return out


def kernel(x, edge_index, pseudo, batch,
           conv1_weight, conv1_root, conv1_bias,
           conv2_weight, conv2_root, conv2_bias,
           conv3_weight, conv3_root, conv3_bias,
           fc1_weight, fc1_bias):
    params = {
        "conv1": {"weight": conv1_weight, "root": conv1_root, "bias": conv1_bias},
        "conv2": {"weight": conv2_weight, "root": conv2_root, "bias": conv2_bias},
        "conv3": {"weight": conv3_weight, "root": conv3_root, "bias": conv3_bias},
        "fc1":   {"weight": fc1_weight, "bias": fc1_bias},
    }
    return _forward(params, x, edge_index, pseudo, batch, num_graphs=64)


# speedup vs baseline: 4.7363x; 1.6774x over previous
import functools

import jax
import jax.numpy as jnp
from jax.experimental import pallas as pl
from jax.experimental.pallas import tpu as pltpu

KS = 5
KT = KS * KS
KPAD = 32
HR = 72
TE = 512
CORES = 2
DEG_ROW = 64
ONE_ROW = 64
VMEM_LIMIT = 48 * 1024 * 1024


def _round_up(x, m):
    return ((x + m - 1) // m) * m


def _layer_kernel(src3_ref, idx_ref, basisT_ref, hT_ref, wT_ref, o_ref,
                  xskT_ref, *, cin, kc, kc_pad, n_pad):
    e = pl.program_id(1)

    @pl.when(e == 0)
    def _():
        o_ref[...] = jnp.zeros_like(o_ref)
        if kc_pad > kc:
            xskT_ref[kc:, :] = jnp.zeros((kc_pad - kc, TE), xskT_ref.dtype)

    src = src3_ref[0]
    rows_n = jax.lax.broadcasted_iota(jnp.int32, (n_pad, TE), 0)
    ohs = (rows_n == src).astype(jnp.bfloat16)
    hsT = jnp.dot(hT_ref[:cin, :], ohs,
                  preferred_element_type=jnp.float32).astype(jnp.bfloat16)

    basisT = basisT_ref[...]
    for k in range(KT):
        xskT_ref[k * cin:(k + 1) * cin, :] = basisT[k:k + 1, :] * hsT

    m = jnp.dot(wT_ref[...], xskT_ref[...],
                preferred_element_type=jnp.float32)
    rows_c = jax.lax.broadcasted_iota(jnp.int32, (HR, TE), 0)
    msgsT = (m + (rows_c == DEG_ROW).astype(jnp.float32)).astype(jnp.bfloat16)

    dstc = idx_ref[0][:, 0:1]
    lanes = jax.lax.broadcasted_iota(jnp.int32, (TE, n_pad), 1)
    ohd = (lanes == dstc).astype(jnp.bfloat16)
    o_ref[0] += jnp.dot(msgsT, ohd, preferred_element_type=jnp.float32)


def _finishT(acc0, acc1, hT, rootT, cout):
    s = acc0 + acc1
    inv = 1.0 / jnp.maximum(s[DEG_ROW:DEG_ROW + 1, :], 1.0)
    a = s * inv + jnp.dot(rootT, hT, preferred_element_type=jnp.float32)
    elu = jnp.where(a > 0.0, a, jnp.exp(jnp.minimum(a, 0.0)) - 1.0)
    rows = jax.lax.broadcasted_iota(jnp.int32, elu.shape, 0)
    out = jnp.where(rows < cout, elu, 0.0)
    return jnp.where(rows == ONE_ROW, 1.0, out)


def _combine_kernel(acc_ref, hT_ref, rootT_ref, o_ref, *, cout):
    o_ref[...] = _finishT(acc_ref[0], acc_ref[1], hT_ref[...],
                          rootT_ref[...], cout).astype(jnp.bfloat16)


def _finale_kernel(acc_ref, hT_ref, rootT_ref, poolT_ref, wfc_ref, bfc_ref,
                   o_ref, *, cout):
    h3 = _finishT(acc_ref[0], acc_ref[1], hT_ref[...], rootT_ref[...],
                  cout).astype(jnp.bfloat16)
    pooledT = jnp.dot(h3, poolT_ref[...],
                      preferred_element_type=jnp.float32)
    logits = jax.lax.dot_general(
        pooledT.astype(jnp.bfloat16), wfc_ref[...],
        (((0,), (0,)), ((), ())),
        preferred_element_type=jnp.float32) + bfc_ref[...]
    mx = jnp.max(logits, axis=1, keepdims=True)
    z = logits - mx
    o_ref[...] = z - jnp.log(jnp.sum(jnp.exp(z), axis=1, keepdims=True))


_VMEM_FULL = pl.BlockSpec(memory_space=pltpu.MemorySpace.VMEM)


def _accumulate(hT, basisT, src3, idx_pack, weight, cin, n_pad, e_pad):
    cout = weight.shape[2]
    kc = KT * cin
    kc_pad = _round_up(kc, 8)
    wT = jnp.pad(weight.reshape(kc, cout).T,
                 ((0, HR - cout), (0, kc_pad - kc))).astype(jnp.bfloat16)
    etc = e_pad // TE // CORES

    return pl.pallas_call(
        functools.partial(_layer_kernel, cin=cin, kc=kc, kc_pad=kc_pad,
                          n_pad=n_pad),
        out_shape=jax.ShapeDtypeStruct((CORES, HR, n_pad), jnp.float32),
        grid=(CORES, etc),
        in_specs=[
            pl.BlockSpec((1, 1, TE), lambda c, e: (c * etc + e, 0, 0)),
            pl.BlockSpec((1, TE, 8), lambda c, e: (c * etc + e, 0, 0)),
            pl.BlockSpec((KPAD, TE), lambda c, e: (0, c * etc + e)),
            pl.BlockSpec((HR, n_pad), lambda c, e: (0, 0)),
            pl.BlockSpec((HR, kc_pad), lambda c, e: (0, 0)),
        ],
        out_specs=pl.BlockSpec((1, HR, n_pad), lambda c, e: (c, 0, 0)),
        scratch_shapes=[pltpu.VMEM((kc_pad, TE), jnp.bfloat16)],
        compiler_params=pltpu.CompilerParams(
            dimension_semantics=("parallel", "arbitrary"),
            vmem_limit_bytes=VMEM_LIMIT),
    )(src3, idx_pack, basisT, hT, wT)


def _root_aug(root, bias, cin, cout):
    rootT = jnp.zeros((HR, HR), jnp.float32)
    rootT = rootT.at[:cout, :cin].set(root.T)
    rootT = rootT.at[:cout, ONE_ROW].set(bias[0])
    return rootT.astype(jnp.bfloat16)


def _combine(acc, hT, root, bias, cin, cout, n_pad):
    hn = n_pad // CORES
    return pl.pallas_call(
        functools.partial(_combine_kernel, cout=cout),
        out_shape=jax.ShapeDtypeStruct((HR, n_pad), jnp.bfloat16),
        grid=(CORES,),
        in_specs=[
            pl.BlockSpec((CORES, HR, hn), lambda i: (0, 0, i)),
            pl.BlockSpec((HR, hn), lambda i: (0, i)),
            pl.BlockSpec((HR, HR), lambda i: (0, 0)),
        ],
        out_specs=pl.BlockSpec((HR, hn), lambda i: (0, i)),
        compiler_params=pltpu.CompilerParams(
            dimension_semantics=("parallel",),
            vmem_limit_bytes=VMEM_LIMIT),
    )(acc, hT, _root_aug(root, bias, cin, cout))


def _finale(acc, hT, root, bias, poolT, w_fc, b_fc, cin, cout):
    w_fc_p = jnp.pad(w_fc, ((0, HR - w_fc.shape[0]), (0, 0))).astype(jnp.bfloat16)
    G = poolT.shape[1]
    n_cls = w_fc.shape[1]
    return pl.pallas_call(
        functools.partial(_finale_kernel, cout=cout),
        out_shape=jax.ShapeDtypeStruct((G, n_cls), jnp.float32),
        in_specs=[_VMEM_FULL] * 6,
        out_specs=_VMEM_FULL,
        compiler_params=pltpu.CompilerParams(vmem_limit_bytes=VMEM_LIMIT),
    )(acc, hT, _root_aug(root, bias, cin, cout), poolT, w_fc_p, b_fc)


def _spline_basis(pseudo):
    v = jnp.clip(pseudo.astype(jnp.float32), 0.0, 1.0) * (KS - 1)
    k0 = jnp.clip(jnp.floor(v), 0.0, KS - 2)
    frac = v - k0
    k0 = k0.astype(jnp.int32)
    B = jnp.zeros((pseudo.shape[0], KT), jnp.float32)
    for s0 in (0, 1):
        for s1 in (0, 1):
            c0 = frac[:, 0] if s0 else (1.0 - frac[:, 0])
            c1 = frac[:, 1] if s1 else (1.0 - frac[:, 1])
            idx = (k0[:, 0] + s0) + KS * (k0[:, 1] + s1)
            B = B + (c0 * c1)[:, None] * jax.nn.one_hot(
                idx, KT, dtype=jnp.float32)
    return B


@functools.partial(jax.jit, static_argnames=("num_graphs",))
def _forward(params, x, edge_index, pseudo, batch, num_graphs):
    N = x.shape[0]
    E = edge_index.shape[1]
    src, dst = edge_index[0], edge_index[1]

    n_pad = _round_up(N, 128 * CORES)
    e_pad = _round_up(E, TE * CORES)

    basisT = jnp.pad(_spline_basis(pseudo),
                     ((0, e_pad - E), (0, KPAD - KT))).astype(jnp.bfloat16).T

    dst_p = jnp.concatenate([dst, jnp.full((e_pad - E,), -1, jnp.int32)])
    src_p = jnp.concatenate([src, jnp.zeros((e_pad - E,), jnp.int32)])
    src3 = src_p.reshape(e_pad // TE, 1, TE)
    idx_pack = jnp.zeros((e_pad, 8), jnp.int32).at[:, 0].set(dst_p)
    idx_pack = idx_pack.reshape(e_pad // TE, TE, 8)

    g_ids = jax.lax.broadcasted_iota(jnp.int32, (num_graphs, N), 0)
    pool = (g_ids == batch[None, :]).astype(jnp.float32)
    pool = pool / jnp.maximum(jnp.sum(pool, axis=1, keepdims=True), 1.0)
    poolT = jnp.pad(pool, ((0, 0), (0, n_pad - N))).astype(jnp.bfloat16).T

    hT = jnp.zeros((HR, n_pad), jnp.bfloat16)
    hT = hT.at[:x.shape[1], :N].set(x.T.astype(jnp.bfloat16))
    hT = hT.at[ONE_ROW, :].set(jnp.bfloat16(1.0))

    dims = (("conv1", 8, 32), ("conv2", 32, 64), ("conv3", 64, 64))
    out = None
    for name, cin, cout in dims:
        p = params[name]
        acc = _accumulate(hT, basisT, src3, idx_pack, p["weight"],
                          cin, n_pad, e_pad)
        if name != "conv3":
            hT = _combine(acc, hT, p["root"], p["bias"], cin, cout, n_pad)
        else:
            out = _finale(acc, hT, p["root"], p["bias"], poolT,
                          params["fc1"]["weight"], params["fc1"]["bias"],
                          cin, cout)
    return out


def kernel(x, edge_index, pseudo, batch,
           conv1_weight, conv1_root, conv1_bias,
           conv2_weight, conv2_root, conv2_bias,
           conv3_weight, conv3_root, conv3_bias,
           fc1_weight, fc1_bias):
    params = {
        "conv1": {"weight": conv1_weight, "root": conv1_root, "bias": conv1_bias},
        "conv2": {"weight": conv2_weight, "root": conv2_root, "bias": conv2_bias},
        "conv3": {"weight": conv3_weight, "root": conv3_root, "bias": conv3_bias},
        "fc1":   {"weight": fc1_weight, "bias": fc1_bias},
    }
    return _forward(params, x, edge_index, pseudo, batch, num_graphs=64)


# speedup vs baseline: 6.1860x; 1.3061x over previous
import functools

import jax
import jax.numpy as jnp
from jax.experimental import pallas as pl
from jax.experimental.pallas import tpu as pltpu

KS = 5
KT = KS * KS
KPAD = 32
HR = 72
TE = 1024
CORES = 2
DEG_ROW = 64
ONE_ROW = 64
VMEM_LIMIT = 48 * 1024 * 1024


def _round_up(x, m):
    return ((x + m - 1) // m) * m


def _layer_kernel(src3_ref, idx_ref, basisT_ref, hT_ref, wT_ref, o_ref,
                  xskT_ref, *, cin, kc, kc_pad, n_pad):
    e = pl.program_id(1)

    @pl.when(e == 0)
    def _():
        o_ref[...] = jnp.zeros_like(o_ref)
        if kc_pad > kc:
            xskT_ref[kc:, :] = jnp.zeros((kc_pad - kc, TE), xskT_ref.dtype)

    src = src3_ref[0]
    rows_n = jax.lax.broadcasted_iota(jnp.int32, (n_pad, TE), 0)
    ohs = (rows_n == src).astype(jnp.bfloat16)
    hsT = jnp.dot(hT_ref[:cin, :], ohs,
                  preferred_element_type=jnp.float32).astype(jnp.bfloat16)

    basisT = basisT_ref[...]
    for k in range(KT):
        xskT_ref[k * cin:(k + 1) * cin, :] = basisT[k:k + 1, :] * hsT

    m = jnp.dot(wT_ref[...], xskT_ref[...],
                preferred_element_type=jnp.float32)
    rows_c = jax.lax.broadcasted_iota(jnp.int32, (HR, TE), 0)
    msgsT = (m + (rows_c == DEG_ROW).astype(jnp.float32)).astype(jnp.bfloat16)

    dstc = idx_ref[0][:, 0:1]
    lanes = jax.lax.broadcasted_iota(jnp.int32, (TE, n_pad), 1)
    ohd = (lanes == dstc).astype(jnp.bfloat16)
    o_ref[0] += jnp.dot(msgsT, ohd, preferred_element_type=jnp.float32)


def _finishT(acc0, acc1, hT, rootT, cout):
    s = acc0 + acc1
    inv = 1.0 / jnp.maximum(s[DEG_ROW:DEG_ROW + 1, :], 1.0)
    a = s * inv + jnp.dot(rootT, hT, preferred_element_type=jnp.float32)
    elu = jnp.where(a > 0.0, a, jnp.exp(jnp.minimum(a, 0.0)) - 1.0)
    rows = jax.lax.broadcasted_iota(jnp.int32, elu.shape, 0)
    out = jnp.where(rows < cout, elu, 0.0)
    return jnp.where(rows == ONE_ROW, 1.0, out)


def _combine_kernel(acc_ref, hT_ref, rootT_ref, o_ref, *, cout):
    o_ref[...] = _finishT(acc_ref[0], acc_ref[1], hT_ref[...],
                          rootT_ref[...], cout).astype(jnp.bfloat16)


def _finale_kernel(acc_ref, hT_ref, rootT_ref, poolT_ref, wfc_ref, bfc_ref,
                   o_ref, *, cout):
    h3 = _finishT(acc_ref[0], acc_ref[1], hT_ref[...], rootT_ref[...],
                  cout).astype(jnp.bfloat16)
    pooledT = jnp.dot(h3, poolT_ref[...],
                      preferred_element_type=jnp.float32)
    logits = jax.lax.dot_general(
        pooledT.astype(jnp.bfloat16), wfc_ref[...],
        (((0,), (0,)), ((), ())),
        preferred_element_type=jnp.float32) + bfc_ref[...]
    mx = jnp.max(logits, axis=1, keepdims=True)
    z = logits - mx
    o_ref[...] = z - jnp.log(jnp.sum(jnp.exp(z), axis=1, keepdims=True))


_VMEM_FULL = pl.BlockSpec(memory_space=pltpu.MemorySpace.VMEM)


def _accumulate(hT, basisT, src3, idx_pack, weight, cin, n_pad, e_pad):
    cout = weight.shape[2]
    kc = KT * cin
    kc_pad = _round_up(kc, 8)
    wT = jnp.pad(weight.reshape(kc, cout).T,
                 ((0, HR - cout), (0, kc_pad - kc))).astype(jnp.bfloat16)
    etc = e_pad // TE // CORES

    return pl.pallas_call(
        functools.partial(_layer_kernel, cin=cin, kc=kc, kc_pad=kc_pad,
                          n_pad=n_pad),
        out_shape=jax.ShapeDtypeStruct((CORES, HR, n_pad), jnp.float32),
        grid=(CORES, etc),
        in_specs=[
            pl.BlockSpec((1, 1, TE), lambda c, e: (c * etc + e, 0, 0)),
            pl.BlockSpec((1, TE, 8), lambda c, e: (c * etc + e, 0, 0)),
            pl.BlockSpec((KPAD, TE), lambda c, e: (0, c * etc + e)),
            pl.BlockSpec((HR, n_pad), lambda c, e: (0, 0)),
            pl.BlockSpec((HR, kc_pad), lambda c, e: (0, 0)),
        ],
        out_specs=pl.BlockSpec((1, HR, n_pad), lambda c, e: (c, 0, 0)),
        scratch_shapes=[pltpu.VMEM((kc_pad, TE), jnp.bfloat16)],
        compiler_params=pltpu.CompilerParams(
            dimension_semantics=("parallel", "arbitrary"),
            vmem_limit_bytes=VMEM_LIMIT),
    )(src3, idx_pack, basisT, hT, wT)


def _root_aug(root, bias, cin, cout):
    rootT = jnp.zeros((HR, HR), jnp.float32)
    rootT = rootT.at[:cout, :cin].set(root.T)
    rootT = rootT.at[:cout, ONE_ROW].set(bias[0])
    return rootT.astype(jnp.bfloat16)


def _combine(acc, hT, root, bias, cin, cout, n_pad):
    hn = n_pad // CORES
    return pl.pallas_call(
        functools.partial(_combine_kernel, cout=cout),
        out_shape=jax.ShapeDtypeStruct((HR, n_pad), jnp.bfloat16),
        grid=(CORES,),
        in_specs=[
            pl.BlockSpec((CORES, HR, hn), lambda i: (0, 0, i)),
            pl.BlockSpec((HR, hn), lambda i: (0, i)),
            pl.BlockSpec((HR, HR), lambda i: (0, 0)),
        ],
        out_specs=pl.BlockSpec((HR, hn), lambda i: (0, i)),
        compiler_params=pltpu.CompilerParams(
            dimension_semantics=("parallel",),
            vmem_limit_bytes=VMEM_LIMIT),
    )(acc, hT, _root_aug(root, bias, cin, cout))


def _finale(acc, hT, root, bias, poolT, w_fc, b_fc, cin, cout):
    w_fc_p = jnp.pad(w_fc, ((0, HR - w_fc.shape[0]), (0, 0))).astype(jnp.bfloat16)
    G = poolT.shape[1]
    n_cls = w_fc.shape[1]
    return pl.pallas_call(
        functools.partial(_finale_kernel, cout=cout),
        out_shape=jax.ShapeDtypeStruct((G, n_cls), jnp.float32),
        in_specs=[_VMEM_FULL] * 6,
        out_specs=_VMEM_FULL,
        compiler_params=pltpu.CompilerParams(vmem_limit_bytes=VMEM_LIMIT),
    )(acc, hT, _root_aug(root, bias, cin, cout), poolT, w_fc_p, b_fc)


def _spline_basis(pseudo):
    v = jnp.clip(pseudo.astype(jnp.float32), 0.0, 1.0) * (KS - 1)
    k0 = jnp.clip(jnp.floor(v), 0.0, KS - 2)
    frac = v - k0
    k0 = k0.astype(jnp.int32)
    B = jnp.zeros((pseudo.shape[0], KT), jnp.float32)
    for s0 in (0, 1):
        for s1 in (0, 1):
            c0 = frac[:, 0] if s0 else (1.0 - frac[:, 0])
            c1 = frac[:, 1] if s1 else (1.0 - frac[:, 1])
            idx = (k0[:, 0] + s0) + KS * (k0[:, 1] + s1)
            B = B + (c0 * c1)[:, None] * jax.nn.one_hot(
                idx, KT, dtype=jnp.float32)
    return B


@functools.partial(jax.jit, static_argnames=("num_graphs",))
def _forward(params, x, edge_index, pseudo, batch, num_graphs):
    N = x.shape[0]
    E = edge_index.shape[1]
    src, dst = edge_index[0], edge_index[1]

    n_pad = _round_up(N, 128 * CORES)
    e_pad = _round_up(E, TE * CORES)

    basisT = jnp.pad(_spline_basis(pseudo),
                     ((0, e_pad - E), (0, KPAD - KT))).astype(jnp.bfloat16).T

    dst_p = jnp.concatenate([dst, jnp.full((e_pad - E,), -1, jnp.int32)])
    src_p = jnp.concatenate([src, jnp.zeros((e_pad - E,), jnp.int32)])
    src3 = src_p.reshape(e_pad // TE, 1, TE)
    idx_pack = jnp.zeros((e_pad, 8), jnp.int32).at[:, 0].set(dst_p)
    idx_pack = idx_pack.reshape(e_pad // TE, TE, 8)

    g_ids = jax.lax.broadcasted_iota(jnp.int32, (num_graphs, N), 0)
    pool = (g_ids == batch[None, :]).astype(jnp.float32)
    pool = pool / jnp.maximum(jnp.sum(pool, axis=1, keepdims=True), 1.0)
    poolT = jnp.pad(pool, ((0, 0), (0, n_pad - N))).astype(jnp.bfloat16).T

    hT = jnp.zeros((HR, n_pad), jnp.bfloat16)
    hT = hT.at[:x.shape[1], :N].set(x.T.astype(jnp.bfloat16))
    hT = hT.at[ONE_ROW, :].set(jnp.bfloat16(1.0))

    dims = (("conv1", 8, 32), ("conv2", 32, 64), ("conv3", 64, 64))
    out = None
    for name, cin, cout in dims:
        p = params[name]
        acc = _accumulate(hT, basisT, src3, idx_pack, p["weight"],
                          cin, n_pad, e_pad)
        if name != "conv3":
            hT = _combine(acc, hT, p["root"], p["bias"], cin, cout, n_pad)
        else:
            out = _finale(acc, hT, p["root"], p["bias"], poolT,
                          params["fc1"]["weight"], params["fc1"]["bias"],
                          cin, cout)
    return out


def kernel(x, edge_index, pseudo, batch,
           conv1_weight, conv1_root, conv1_bias,
           conv2_weight, conv2_root, conv2_bias,
           conv3_weight, conv3_root, conv3_bias,
           fc1_weight, fc1_bias):
    params = {
        "conv1": {"weight": conv1_weight, "root": conv1_root, "bias": conv1_bias},
        "conv2": {"weight": conv2_weight, "root": conv2_root, "bias": conv2_bias},
        "conv3": {"weight": conv3_weight, "root": conv3_root, "bias": conv3_bias},
        "fc1":   {"weight": fc1_weight, "bias": fc1_bias},
    }
    return _forward(params, x, edge_index, pseudo, batch, num_graphs=64)


# speedup vs baseline: 6.5851x; 1.0645x over previous
import functools

import jax
import jax.numpy as jnp
from jax.experimental import pallas as pl
from jax.experimental.pallas import tpu as pltpu

KS = 5
KT = KS * KS
KPAD = 32
HR = 72
TE = 2048
CORES = 2
DEG_ROW = 64
ONE_ROW = 64
VMEM_LIMIT = 48 * 1024 * 1024


def _round_up(x, m):
    return ((x + m - 1) // m) * m


def _layer_kernel(src3_ref, idx_ref, basisT_ref, hT_ref, wT_ref, o_ref,
                  xskT_ref, *, cin, kc, kc_pad, n_pad):
    e = pl.program_id(1)

    @pl.when(e == 0)
    def _():
        o_ref[...] = jnp.zeros_like(o_ref)
        if kc_pad > kc:
            xskT_ref[kc:, :] = jnp.zeros((kc_pad - kc, TE), xskT_ref.dtype)

    src = src3_ref[0]
    rows_n = jax.lax.broadcasted_iota(jnp.int32, (n_pad, TE), 0)
    ohs = (rows_n == src).astype(jnp.bfloat16)
    hsT = jnp.dot(hT_ref[:cin, :], ohs,
                  preferred_element_type=jnp.float32).astype(jnp.bfloat16)

    basisT = basisT_ref[...]
    for k in range(KT):
        xskT_ref[k * cin:(k + 1) * cin, :] = basisT[k:k + 1, :] * hsT

    m = jnp.dot(wT_ref[...], xskT_ref[...],
                preferred_element_type=jnp.float32)
    rows_c = jax.lax.broadcasted_iota(jnp.int32, (HR, TE), 0)
    msgsT = (m + (rows_c == DEG_ROW).astype(jnp.float32)).astype(jnp.bfloat16)

    dstc = idx_ref[0][:, 0:1]
    lanes = jax.lax.broadcasted_iota(jnp.int32, (TE, n_pad), 1)
    ohd = (lanes == dstc).astype(jnp.bfloat16)
    o_ref[0] += jnp.dot(msgsT, ohd, preferred_element_type=jnp.float32)


def _finishT(acc0, acc1, hT, rootT, cout):
    s = acc0 + acc1
    inv = 1.0 / jnp.maximum(s[DEG_ROW:DEG_ROW + 1, :], 1.0)
    a = s * inv + jnp.dot(rootT, hT, preferred_element_type=jnp.float32)
    elu = jnp.where(a > 0.0, a, jnp.exp(jnp.minimum(a, 0.0)) - 1.0)
    rows = jax.lax.broadcasted_iota(jnp.int32, elu.shape, 0)
    out = jnp.where(rows < cout, elu, 0.0)
    return jnp.where(rows == ONE_ROW, 1.0, out)


def _combine_kernel(acc_ref, hT_ref, rootT_ref, o_ref, *, cout):
    o_ref[...] = _finishT(acc_ref[0], acc_ref[1], hT_ref[...],
                          rootT_ref[...], cout).astype(jnp.bfloat16)


def _finale_kernel(acc_ref, hT_ref, rootT_ref, poolT_ref, wfc_ref, bfc_ref,
                   o_ref, *, cout):
    h3 = _finishT(acc_ref[0], acc_ref[1], hT_ref[...], rootT_ref[...],
                  cout).astype(jnp.bfloat16)
    pooledT = jnp.dot(h3, poolT_ref[...],
                      preferred_element_type=jnp.float32)
    logits = jax.lax.dot_general(
        pooledT.astype(jnp.bfloat16), wfc_ref[...],
        (((0,), (0,)), ((), ())),
        preferred_element_type=jnp.float32) + bfc_ref[...]
    mx = jnp.max(logits, axis=1, keepdims=True)
    z = logits - mx
    o_ref[...] = z - jnp.log(jnp.sum(jnp.exp(z), axis=1, keepdims=True))


_VMEM_FULL = pl.BlockSpec(memory_space=pltpu.MemorySpace.VMEM)


def _accumulate(hT, basisT, src3, idx_pack, weight, cin, n_pad, e_pad):
    cout = weight.shape[2]
    kc = KT * cin
    kc_pad = _round_up(kc, 8)
    wT = jnp.pad(weight.reshape(kc, cout).T,
                 ((0, HR - cout), (0, kc_pad - kc))).astype(jnp.bfloat16)
    etc = e_pad // TE // CORES

    return pl.pallas_call(
        functools.partial(_layer_kernel, cin=cin, kc=kc, kc_pad=kc_pad,
                          n_pad=n_pad),
        out_shape=jax.ShapeDtypeStruct((CORES, HR, n_pad), jnp.float32),
        grid=(CORES, etc),
        in_specs=[
            pl.BlockSpec((1, 1, TE), lambda c, e: (c * etc + e, 0, 0)),
            pl.BlockSpec((1, TE, 8), lambda c, e: (c * etc + e, 0, 0)),
            pl.BlockSpec((KPAD, TE), lambda c, e: (0, c * etc + e)),
            pl.BlockSpec((HR, n_pad), lambda c, e: (0, 0)),
            pl.BlockSpec((HR, kc_pad), lambda c, e: (0, 0)),
        ],
        out_specs=pl.BlockSpec((1, HR, n_pad), lambda c, e: (c, 0, 0)),
        scratch_shapes=[pltpu.VMEM((kc_pad, TE), jnp.bfloat16)],
        compiler_params=pltpu.CompilerParams(
            dimension_semantics=("parallel", "arbitrary"),
            vmem_limit_bytes=VMEM_LIMIT),
    )(src3, idx_pack, basisT, hT, wT)


def _root_aug(root, bias, cin, cout):
    rootT = jnp.zeros((HR, HR), jnp.float32)
    rootT = rootT.at[:cout, :cin].set(root.T)
    rootT = rootT.at[:cout, ONE_ROW].set(bias[0])
    return rootT.astype(jnp.bfloat16)


def _combine(acc, hT, root, bias, cin, cout, n_pad):
    hn = n_pad // CORES
    return pl.pallas_call(
        functools.partial(_combine_kernel, cout=cout),
        out_shape=jax.ShapeDtypeStruct((HR, n_pad), jnp.bfloat16),
        grid=(CORES,),
        in_specs=[
            pl.BlockSpec((CORES, HR, hn), lambda i: (0, 0, i)),
            pl.BlockSpec((HR, hn), lambda i: (0, i)),
            pl.BlockSpec((HR, HR), lambda i: (0, 0)),
        ],
        out_specs=pl.BlockSpec((HR, hn), lambda i: (0, i)),
        compiler_params=pltpu.CompilerParams(
            dimension_semantics=("parallel",),
            vmem_limit_bytes=VMEM_LIMIT),
    )(acc, hT, _root_aug(root, bias, cin, cout))


def _finale(acc, hT, root, bias, poolT, w_fc, b_fc, cin, cout):
    w_fc_p = jnp.pad(w_fc, ((0, HR - w_fc.shape[0]), (0, 0))).astype(jnp.bfloat16)
    G = poolT.shape[1]
    n_cls = w_fc.shape[1]
    return pl.pallas_call(
        functools.partial(_finale_kernel, cout=cout),
        out_shape=jax.ShapeDtypeStruct((G, n_cls), jnp.float32),
        in_specs=[_VMEM_FULL] * 6,
        out_specs=_VMEM_FULL,
        compiler_params=pltpu.CompilerParams(vmem_limit_bytes=VMEM_LIMIT),
    )(acc, hT, _root_aug(root, bias, cin, cout), poolT, w_fc_p, b_fc)


def _spline_basis(pseudo):
    v = jnp.clip(pseudo.astype(jnp.float32), 0.0, 1.0) * (KS - 1)
    k0 = jnp.clip(jnp.floor(v), 0.0, KS - 2)
    frac = v - k0
    k0 = k0.astype(jnp.int32)
    B = jnp.zeros((pseudo.shape[0], KT), jnp.float32)
    for s0 in (0, 1):
        for s1 in (0, 1):
            c0 = frac[:, 0] if s0 else (1.0 - frac[:, 0])
            c1 = frac[:, 1] if s1 else (1.0 - frac[:, 1])
            idx = (k0[:, 0] + s0) + KS * (k0[:, 1] + s1)
            B = B + (c0 * c1)[:, None] * jax.nn.one_hot(
                idx, KT, dtype=jnp.float32)
    return B


@functools.partial(jax.jit, static_argnames=("num_graphs",))
def _forward(params, x, edge_index, pseudo, batch, num_graphs):
    N = x.shape[0]
    E = edge_index.shape[1]
    src, dst = edge_index[0], edge_index[1]

    n_pad = _round_up(N, 128 * CORES)
    e_pad = _round_up(E, TE * CORES)

    basisT = jnp.pad(_spline_basis(pseudo),
                     ((0, e_pad - E), (0, KPAD - KT))).astype(jnp.bfloat16).T

    dst_p = jnp.concatenate([dst, jnp.full((e_pad - E,), -1, jnp.int32)])
    src_p = jnp.concatenate([src, jnp.zeros((e_pad - E,), jnp.int32)])
    src3 = src_p.reshape(e_pad // TE, 1, TE)
    idx_pack = jnp.zeros((e_pad, 8), jnp.int32).at[:, 0].set(dst_p)
    idx_pack = idx_pack.reshape(e_pad // TE, TE, 8)

    g_ids = jax.lax.broadcasted_iota(jnp.int32, (num_graphs, N), 0)
    pool = (g_ids == batch[None, :]).astype(jnp.float32)
    pool = pool / jnp.maximum(jnp.sum(pool, axis=1, keepdims=True), 1.0)
    poolT = jnp.pad(pool, ((0, 0), (0, n_pad - N))).astype(jnp.bfloat16).T

    hT = jnp.zeros((HR, n_pad), jnp.bfloat16)
    hT = hT.at[:x.shape[1], :N].set(x.T.astype(jnp.bfloat16))
    hT = hT.at[ONE_ROW, :].set(jnp.bfloat16(1.0))

    dims = (("conv1", 8, 32), ("conv2", 32, 64), ("conv3", 64, 64))
    out = None
    for name, cin, cout in dims:
        p = params[name]
        acc = _accumulate(hT, basisT, src3, idx_pack, p["weight"],
                          cin, n_pad, e_pad)
        if name != "conv3":
            hT = _combine(acc, hT, p["root"], p["bias"], cin, cout, n_pad)
        else:
            out = _finale(acc, hT, p["root"], p["bias"], poolT,
                          params["fc1"]["weight"], params["fc1"]["bias"],
                          cin, cout)
    return out


def kernel(x, edge_index, pseudo, batch,
           conv1_weight, conv1_root, conv1_bias,
           conv2_weight, conv2_root, conv2_bias,
           conv3_weight, conv3_root, conv3_bias,
           fc1_weight, fc1_bias):
    params = {
        "conv1": {"weight": conv1_weight, "root": conv1_root, "bias": conv1_bias},
        "conv2": {"weight": conv2_weight, "root": conv2_root, "bias": conv2_bias},
        "conv3": {"weight": conv3_weight, "root": conv3_root, "bias": conv3_bias},
        "fc1":   {"weight": fc1_weight, "bias": fc1_bias},
    }
    return _forward(params, x, edge_index, pseudo, batch, num_graphs=64)


# speedup vs baseline: 6.7664x; 1.0275x over previous
import functools

import jax
import jax.numpy as jnp
from jax.experimental import pallas as pl
from jax.experimental.pallas import tpu as pltpu

KS = 5
KT = KS * KS
KPAD = 32
HR = 72
TE = 4096
CORES = 2
DEG_ROW = 64
ONE_ROW = 64
VMEM_LIMIT = 48 * 1024 * 1024


def _round_up(x, m):
    return ((x + m - 1) // m) * m


def _layer_kernel(src3_ref, idx_ref, basisT_ref, hT_ref, wT_ref, o_ref,
                  xskT_ref, *, cin, kc, kc_pad, n_pad):
    e = pl.program_id(1)

    @pl.when(e == 0)
    def _():
        o_ref[...] = jnp.zeros_like(o_ref)
        if kc_pad > kc:
            xskT_ref[kc:, :] = jnp.zeros((kc_pad - kc, TE), xskT_ref.dtype)

    src = src3_ref[0]
    rows_n = jax.lax.broadcasted_iota(jnp.int32, (n_pad, TE), 0)
    ohs = (rows_n == src).astype(jnp.bfloat16)
    hsT = jnp.dot(hT_ref[:cin, :], ohs,
                  preferred_element_type=jnp.float32).astype(jnp.bfloat16)

    basisT = basisT_ref[...]
    for k in range(KT):
        xskT_ref[k * cin:(k + 1) * cin, :] = basisT[k:k + 1, :] * hsT

    m = jnp.dot(wT_ref[...], xskT_ref[...],
                preferred_element_type=jnp.float32)
    rows_c = jax.lax.broadcasted_iota(jnp.int32, (HR, TE), 0)
    msgsT = (m + (rows_c == DEG_ROW).astype(jnp.float32)).astype(jnp.bfloat16)

    dstc = idx_ref[0][:, 0:1]
    lanes = jax.lax.broadcasted_iota(jnp.int32, (TE, n_pad), 1)
    ohd = (lanes == dstc).astype(jnp.bfloat16)
    o_ref[0] += jnp.dot(msgsT, ohd, preferred_element_type=jnp.float32)


def _finishT(acc0, acc1, hT, rootT, cout):
    s = acc0 + acc1
    inv = 1.0 / jnp.maximum(s[DEG_ROW:DEG_ROW + 1, :], 1.0)
    a = s * inv + jnp.dot(rootT, hT, preferred_element_type=jnp.float32)
    elu = jnp.where(a > 0.0, a, jnp.exp(jnp.minimum(a, 0.0)) - 1.0)
    rows = jax.lax.broadcasted_iota(jnp.int32, elu.shape, 0)
    out = jnp.where(rows < cout, elu, 0.0)
    return jnp.where(rows == ONE_ROW, 1.0, out)


def _combine_kernel(acc_ref, hT_ref, rootT_ref, o_ref, *, cout):
    o_ref[...] = _finishT(acc_ref[0], acc_ref[1], hT_ref[...],
                          rootT_ref[...], cout).astype(jnp.bfloat16)


def _finale_kernel(acc_ref, hT_ref, rootT_ref, poolT_ref, wfc_ref, bfc_ref,
                   o_ref, *, cout):
    h3 = _finishT(acc_ref[0], acc_ref[1], hT_ref[...], rootT_ref[...],
                  cout).astype(jnp.bfloat16)
    pooledT = jnp.dot(h3, poolT_ref[...],
                      preferred_element_type=jnp.float32)
    logits = jax.lax.dot_general(
        pooledT.astype(jnp.bfloat16), wfc_ref[...],
        (((0,), (0,)), ((), ())),
        preferred_element_type=jnp.float32) + bfc_ref[...]
    mx = jnp.max(logits, axis=1, keepdims=True)
    z = logits - mx
    o_ref[...] = z - jnp.log(jnp.sum(jnp.exp(z), axis=1, keepdims=True))


_VMEM_FULL = pl.BlockSpec(memory_space=pltpu.MemorySpace.VMEM)


def _accumulate(hT, basisT, src3, idx_pack, weight, cin, n_pad, e_pad):
    cout = weight.shape[2]
    kc = KT * cin
    kc_pad = _round_up(kc, 8)
    wT = jnp.pad(weight.reshape(kc, cout).T,
                 ((0, HR - cout), (0, kc_pad - kc))).astype(jnp.bfloat16)
    etc = e_pad // TE // CORES

    return pl.pallas_call(
        functools.partial(_layer_kernel, cin=cin, kc=kc, kc_pad=kc_pad,
                          n_pad=n_pad),
        out_shape=jax.ShapeDtypeStruct((CORES, HR, n_pad), jnp.float32),
        grid=(CORES, etc),
        in_specs=[
            pl.BlockSpec((1, 1, TE), lambda c, e: (c * etc + e, 0, 0)),
            pl.BlockSpec((1, TE, 8), lambda c, e: (c * etc + e, 0, 0)),
            pl.BlockSpec((KPAD, TE), lambda c, e: (0, c * etc + e)),
            pl.BlockSpec((HR, n_pad), lambda c, e: (0, 0)),
            pl.BlockSpec((HR, kc_pad), lambda c, e: (0, 0)),
        ],
        out_specs=pl.BlockSpec((1, HR, n_pad), lambda c, e: (c, 0, 0)),
        scratch_shapes=[pltpu.VMEM((kc_pad, TE), jnp.bfloat16)],
        compiler_params=pltpu.CompilerParams(
            dimension_semantics=("parallel", "arbitrary"),
            vmem_limit_bytes=VMEM_LIMIT),
    )(src3, idx_pack, basisT, hT, wT)


def _root_aug(root, bias, cin, cout):
    rootT = jnp.zeros((HR, HR), jnp.float32)
    rootT = rootT.at[:cout, :cin].set(root.T)
    rootT = rootT.at[:cout, ONE_ROW].set(bias[0])
    return rootT.astype(jnp.bfloat16)


def _combine(acc, hT, root, bias, cin, cout, n_pad):
    hn = n_pad // CORES
    return pl.pallas_call(
        functools.partial(_combine_kernel, cout=cout),
        out_shape=jax.ShapeDtypeStruct((HR, n_pad), jnp.bfloat16),
        grid=(CORES,),
        in_specs=[
            pl.BlockSpec((CORES, HR, hn), lambda i: (0, 0, i)),
            pl.BlockSpec((HR, hn), lambda i: (0, i)),
            pl.BlockSpec((HR, HR), lambda i: (0, 0)),
        ],
        out_specs=pl.BlockSpec((HR, hn), lambda i: (0, i)),
        compiler_params=pltpu.CompilerParams(
            dimension_semantics=("parallel",),
            vmem_limit_bytes=VMEM_LIMIT),
    )(acc, hT, _root_aug(root, bias, cin, cout))


def _finale(acc, hT, root, bias, poolT, w_fc, b_fc, cin, cout):
    w_fc_p = jnp.pad(w_fc, ((0, HR - w_fc.shape[0]), (0, 0))).astype(jnp.bfloat16)
    G = poolT.shape[1]
    n_cls = w_fc.shape[1]
    return pl.pallas_call(
        functools.partial(_finale_kernel, cout=cout),
        out_shape=jax.ShapeDtypeStruct((G, n_cls), jnp.float32),
        in_specs=[_VMEM_FULL] * 6,
        out_specs=_VMEM_FULL,
        compiler_params=pltpu.CompilerParams(vmem_limit_bytes=VMEM_LIMIT),
    )(acc, hT, _root_aug(root, bias, cin, cout), poolT, w_fc_p, b_fc)


def _spline_basis(pseudo):
    v = jnp.clip(pseudo.astype(jnp.float32), 0.0, 1.0) * (KS - 1)
    k0 = jnp.clip(jnp.floor(v), 0.0, KS - 2)
    frac = v - k0
    k0 = k0.astype(jnp.int32)
    B = jnp.zeros((pseudo.shape[0], KT), jnp.float32)
    for s0 in (0, 1):
        for s1 in (0, 1):
            c0 = frac[:, 0] if s0 else (1.0 - frac[:, 0])
            c1 = frac[:, 1] if s1 else (1.0 - frac[:, 1])
            idx = (k0[:, 0] + s0) + KS * (k0[:, 1] + s1)
            B = B + (c0 * c1)[:, None] * jax.nn.one_hot(
                idx, KT, dtype=jnp.float32)
    return B


@functools.partial(jax.jit, static_argnames=("num_graphs",))
def _forward(params, x, edge_index, pseudo, batch, num_graphs):
    N = x.shape[0]
    E = edge_index.shape[1]
    src, dst = edge_index[0], edge_index[1]

    n_pad = _round_up(N, 128 * CORES)
    e_pad = _round_up(E, TE * CORES)

    basisT = jnp.pad(_spline_basis(pseudo),
                     ((0, e_pad - E), (0, KPAD - KT))).astype(jnp.bfloat16).T

    dst_p = jnp.concatenate([dst, jnp.full((e_pad - E,), -1, jnp.int32)])
    src_p = jnp.concatenate([src, jnp.zeros((e_pad - E,), jnp.int32)])
    src3 = src_p.reshape(e_pad // TE, 1, TE)
    idx_pack = jnp.zeros((e_pad, 8), jnp.int32).at[:, 0].set(dst_p)
    idx_pack = idx_pack.reshape(e_pad // TE, TE, 8)

    g_ids = jax.lax.broadcasted_iota(jnp.int32, (num_graphs, N), 0)
    pool = (g_ids == batch[None, :]).astype(jnp.float32)
    pool = pool / jnp.maximum(jnp.sum(pool, axis=1, keepdims=True), 1.0)
    poolT = jnp.pad(pool, ((0, 0), (0, n_pad - N))).astype(jnp.bfloat16).T

    hT = jnp.zeros((HR, n_pad), jnp.bfloat16)
    hT = hT.at[:x.shape[1], :N].set(x.T.astype(jnp.bfloat16))
    hT = hT.at[ONE_ROW, :].set(jnp.bfloat16(1.0))

    dims = (("conv1", 8, 32), ("conv2", 32, 64), ("conv3", 64, 64))
    out = None
    for name, cin, cout in dims:
        p = params[name]
        acc = _accumulate(hT, basisT, src3, idx_pack, p["weight"],
                          cin, n_pad, e_pad)
        if name != "conv3":
            hT = _combine(acc, hT, p["root"], p["bias"], cin, cout, n_pad)
        else:
            out = _finale(acc, hT, p["root"], p["bias"], poolT,
                          params["fc1"]["weight"], params["fc1"]["bias"],
                          cin, cout)
    return out


def kernel(x, edge_index, pseudo, batch,
           conv1_weight, conv1_root, conv1_bias,
           conv2_weight, conv2_root, conv2_bias,
           conv3_weight, conv3_root, conv3_bias,
           fc1_weight, fc1_bias):
    params = {
        "conv1": {"weight": conv1_weight, "root": conv1_root, "bias": conv1_bias},
        "conv2": {"weight": conv2_weight, "root": conv2_root, "bias": conv2_bias},
        "conv3": {"weight": conv3_weight, "root": conv3_root, "bias": conv3_bias},
        "fc1":   {"weight": fc1_weight, "bias": fc1_bias},
    }
    return _forward(params, x, edge_index, pseudo, batch, num_graphs=64)
